# baseline (speedup 1.0000x reference)
import jax
import jax.numpy as jnp
import numpy as np
from jax.experimental import pallas as pl
from jax.experimental.pallas import tpu as pltpu

_DIM = 96
_HEADS = 6
_HD = 16
_WS = 16
_L = _WS * _WS
_K1 = _L // 2 - 1
_SCALE = 1.0 / np.sqrt(_HD)
_WPB = 24


def _wmosa_body(xw_ref, rW_ref, wqkv_ref, woT_ref, pW_ref, pb_ref, out_ref):
    FL = _WPB * _L
    NR = _WPB * _HEADS
    jlane = jax.lax.broadcasted_iota(jnp.int32, (1, _L), 1)
    ii = jax.lax.broadcasted_iota(jnp.int32, (_L, _L), 0)
    jj = jax.lax.broadcasted_iota(jnp.int32, (_L, _L), 1)
    LT = (ii < jj).astype(jnp.float32)
    ones_col = jnp.ones((_L, 1), jnp.float32)

    xw_flat = xw_ref[...].reshape(FL, _DIM)
    rawT = jax.lax.dot_general(
        rW_ref[...], xw_flat, (((0,), (1,)), ((), ())),
        preferred_element_type=jnp.float32)
    logitsT = jax.nn.sigmoid(rawT)
    lT = logitsT.reshape(NR, _L)
    qkvT = jax.lax.dot_general(
        wqkv_ref[...], xw_flat, (((0,), (1,)), ((), ())),
        preferred_element_type=jnp.float32)

    v = jnp.where(jlane == 0, -1.0, lT)
    size = 2
    while size <= _L:
        desc = (jlane & size) == 0
        stride = size // 2
        while stride >= 1:
            is_low = (jlane & stride) == 0
            pv = jnp.where(is_low, pltpu.roll(v, _L - stride, 1),
                           pltpu.roll(v, stride, 1))
            keep_max = desc == is_low
            v = jnp.where(keep_max, jnp.maximum(v, pv), jnp.minimum(v, pv))
            stride //= 2
        size *= 2
    tau = v[:, _K1 - 1:_K1]

    tail = jlane >= 1
    gt = (lT > tau) & tail
    eq = (lT == tau) & tail
    n_gt = jnp.dot(gt.astype(jnp.float32), ones_col,
                   preferred_element_type=jnp.float32)
    prefix = jnp.dot(eq.astype(jnp.float32), LT,
                     preferred_element_type=jnp.float32)
    mask = (jlane == 0) | gt | (eq & (prefix < (_K1 - n_gt)))
    maskf = mask.astype(jnp.float32)
    gatef = jnp.where(mask, lT, 0.0)

    pairs = [(w, h) for w in range(_WPB) for h in range(_HEADS)]
    ps, vms = [], []
    for w, h in pairs:
        base = w * _L
        qT = qkvT[h * 48:h * 48 + 16, base:base + _L]
        kT = qkvT[h * 48 + 16:h * 48 + 32, base:base + _L]
        vT = qkvT[h * 48 + 32:h * 48 + 48, base:base + _L]
        r = h * _WPB + w
        mrow = maskf[r:r + 1, :]
        scores = jax.lax.dot_general(
            kT, qT, (((0,), (0,)), ((), ())),
            preferred_element_type=jnp.float32)
        ps.append(jnp.exp(scores.astype(jnp.bfloat16)))
        vms.append(jnp.concatenate([vT * mrow, mrow],
                                   axis=0).astype(jnp.bfloat16))
    oTs = {}
    for i, (w, h) in enumerate(pairs):
        r = h * _WPB + w
        os_ = jax.lax.dot_general(
            vms[i], ps[i], (((1,), (0,)), ((), ())),
            preferred_element_type=jnp.float32)
        oTs[(w, h)] = os_[:_HD] * (gatef[r:r + 1, :] / os_[_HD:_HD + 1])

    ocatT = jnp.concatenate(
        [jnp.concatenate([oTs[(w, h)] for h in range(_HEADS)], axis=0)
         for w in range(_WPB)], axis=1)
    presT = jnp.dot(woT_ref[...], ocatT,
                    preferred_element_type=jnp.float32)
    y = jax.lax.dot_general(
        presT, pW_ref[...], (((0,), (0,)), ((), ())),
        preferred_element_type=jnp.float32) + pb_ref[...]
    out_ref[...] = y.reshape(_WPB, _L, _DIM)


def kernel(x, r_W, W_qkv, W_o, proj_W, proj_b):
    B, H, W, C = x.shape
    nh, nw = H // _WS, W // _WS
    xw = x.reshape(B, nh, _WS, nw, _WS, C).transpose(0, 1, 3, 2, 4, 5)
    xw = xw.reshape(B * nh * nw, _L, C)
    Bn = xw.shape[0]

    qscale = jnp.concatenate([jnp.full((_HD,), _SCALE, jnp.float32),
                              jnp.ones((2 * _HD,), jnp.float32)])
    wqkv = jnp.transpose(W_qkv * qscale, (1, 0, 2)).reshape(C, _HEADS * 3 * _HD)
    woT = W_o.reshape(_HEADS * _HD, C).T
    pb = proj_b.reshape(1, C)

    y = pl.pallas_call(
        _wmosa_body,
        grid=(Bn // _WPB,),
        in_specs=[
            pl.BlockSpec((_WPB, _L, C), lambda i: (i, 0, 0)),
            pl.BlockSpec((C, _HEADS), lambda i: (0, 0)),
            pl.BlockSpec((C, _HEADS * 3 * _HD), lambda i: (0, 0)),
            pl.BlockSpec((C, _HEADS * _HD), lambda i: (0, 0)),
            pl.BlockSpec((C, C), lambda i: (0, 0)),
            pl.BlockSpec((1, C), lambda i: (0, 0)),
        ],
        out_specs=pl.BlockSpec((_WPB, _L, C), lambda i: (i, 0, 0)),
        out_shape=jax.ShapeDtypeStruct((Bn, _L, C), x.dtype),
    )(xw, r_W, wqkv, woT, proj_W, pb)

    x_out = y.reshape(B, nh, nw, _WS, _WS, C).transpose(0, 1, 3, 2, 4, 5)
    return x_out.reshape(B, H, W, C)


# speedup vs baseline: 1.0371x; 1.0371x over previous
import jax
import jax.numpy as jnp
import numpy as np
from jax.experimental import pallas as pl
from jax.experimental.pallas import tpu as pltpu

_DIM = 96
_HEADS = 6
_HD = 16
_WS = 16
_L = _WS * _WS
_K1 = _L // 2 - 1
_SCALE = 1.0 / np.sqrt(_HD)
_WPB = 24


def _wmosa_body(xw_ref, rW_ref, wqkv_ref, woT_ref, pW_ref, pb_ref, out_ref):
    FL = _WPB * _L
    NR = _WPB * _HEADS
    jlane = jax.lax.broadcasted_iota(jnp.int32, (1, _L), 1)
    ii = jax.lax.broadcasted_iota(jnp.int32, (_L, _L), 0)
    jj = jax.lax.broadcasted_iota(jnp.int32, (_L, _L), 1)
    LT = (ii < jj).astype(jnp.float32)
    ones_col = jnp.ones((_L, 1), jnp.float32)

    xw_flat = xw_ref[...].reshape(FL, _DIM)
    rawT = jax.lax.dot_general(
        rW_ref[...], xw_flat, (((0,), (1,)), ((), ())),
        preferred_element_type=jnp.float32)
    logitsT = jax.nn.sigmoid(rawT)
    lT = logitsT.reshape(NR, _L)
    qkvT = jax.lax.dot_general(
        wqkv_ref[...], xw_flat, (((0,), (1,)), ((), ())),
        preferred_element_type=jnp.float32)

    v = jnp.where(jlane == 0, -1.0, lT)
    size = 2
    while size <= _L:
        desc = (jlane & size) == 0
        stride = size // 2
        while stride >= 1:
            is_low = (jlane & stride) == 0
            pv = jnp.where(is_low, pltpu.roll(v, _L - stride, 1),
                           pltpu.roll(v, stride, 1))
            keep_max = desc == is_low
            v = jnp.where(keep_max, jnp.maximum(v, pv), jnp.minimum(v, pv))
            stride //= 2
        size *= 2
    tau = v[:, _K1 - 1:_K1]

    tail = jlane >= 1
    gt = (lT > tau) & tail
    eq = (lT == tau) & tail
    n_gt = jnp.dot(gt.astype(jnp.float32), ones_col,
                   preferred_element_type=jnp.float32)
    prefix = jnp.dot(eq.astype(jnp.float32), LT,
                     preferred_element_type=jnp.float32)
    mask = (jlane == 0) | gt | (eq & (prefix < (_K1 - n_gt)))
    maskf = mask.astype(jnp.float32)
    gatef = jnp.where(mask, lT, 0.0)

    pairs = [(w, h) for w in range(_WPB) for h in range(_HEADS)]
    qkvb = qkvT.astype(jnp.bfloat16)
    maskb = maskf.astype(jnp.bfloat16)
    ps, vms = [], []
    for w, h in pairs:
        base = w * _L
        qT = qkvb[h * 48:h * 48 + 16, base:base + _L]
        kT = qkvb[h * 48 + 16:h * 48 + 32, base:base + _L]
        vT = qkvb[h * 48 + 32:h * 48 + 48, base:base + _L]
        r = h * _WPB + w
        mrow = maskb[r:r + 1, :]
        scores = jax.lax.dot_general(
            kT, qT, (((0,), (0,)), ((), ())),
            preferred_element_type=jnp.float32)
        ps.append(jnp.exp(scores).astype(jnp.bfloat16))
        vms.append(jnp.concatenate([vT * mrow, mrow], axis=0))
    oTs = {}
    for i, (w, h) in enumerate(pairs):
        r = h * _WPB + w
        os_ = jax.lax.dot_general(
            vms[i], ps[i], (((1,), (0,)), ((), ())),
            preferred_element_type=jnp.float32)
        oTs[(w, h)] = os_[:_HD] * (gatef[r:r + 1, :] / os_[_HD:_HD + 1])

    ocatT = jnp.concatenate(
        [jnp.concatenate([oTs[(w, h)] for h in range(_HEADS)], axis=0)
         for w in range(_WPB)], axis=1)
    presT = jnp.dot(woT_ref[...], ocatT,
                    preferred_element_type=jnp.float32)
    y = jax.lax.dot_general(
        presT, pW_ref[...], (((0,), (0,)), ((), ())),
        preferred_element_type=jnp.float32) + pb_ref[...]
    out_ref[...] = y.reshape(_WPB, _L, _DIM)


def kernel(x, r_W, W_qkv, W_o, proj_W, proj_b):
    B, H, W, C = x.shape
    nh, nw = H // _WS, W // _WS
    xw = x.reshape(B, nh, _WS, nw, _WS, C).transpose(0, 1, 3, 2, 4, 5)
    xw = xw.reshape(B * nh * nw, _L, C)
    Bn = xw.shape[0]

    qscale = jnp.concatenate([jnp.full((_HD,), _SCALE, jnp.float32),
                              jnp.ones((2 * _HD,), jnp.float32)])
    wqkv = jnp.transpose(W_qkv * qscale, (1, 0, 2)).reshape(C, _HEADS * 3 * _HD)
    woT = W_o.reshape(_HEADS * _HD, C).T
    pb = proj_b.reshape(1, C)

    y = pl.pallas_call(
        _wmosa_body,
        grid=(Bn // _WPB,),
        in_specs=[
            pl.BlockSpec((_WPB, _L, C), lambda i: (i, 0, 0)),
            pl.BlockSpec((C, _HEADS), lambda i: (0, 0)),
            pl.BlockSpec((C, _HEADS * 3 * _HD), lambda i: (0, 0)),
            pl.BlockSpec((C, _HEADS * _HD), lambda i: (0, 0)),
            pl.BlockSpec((C, C), lambda i: (0, 0)),
            pl.BlockSpec((1, C), lambda i: (0, 0)),
        ],
        out_specs=pl.BlockSpec((_WPB, _L, C), lambda i: (i, 0, 0)),
        out_shape=jax.ShapeDtypeStruct((Bn, _L, C), x.dtype),
    )(xw, r_W, wqkv, woT, proj_W, pb)

    x_out = y.reshape(B, nh, nw, _WS, _WS, C).transpose(0, 1, 3, 2, 4, 5)
    return x_out.reshape(B, H, W, C)


# speedup vs baseline: 1.2975x; 1.2511x over previous
import jax
import jax.numpy as jnp
import numpy as np
from jax.experimental import pallas as pl
from jax.experimental.pallas import tpu as pltpu

_DIM = 96
_HEADS = 6
_HD = 16
_WS = 16
_L = _WS * _WS
_K1 = _L // 2 - 1
_SCALE = 1.0 / np.sqrt(_HD)
_WPB = 24


def _wmosa_body(xw_ref, rW_ref, wqkv_ref, woT_ref, pW_ref, pb_ref, out_ref):
    FL = _WPB * _L
    NR = _WPB * _HEADS
    jlane = jax.lax.broadcasted_iota(jnp.int32, (1, _L), 1)
    ii = jax.lax.broadcasted_iota(jnp.int32, (_L, _L), 0)
    jj = jax.lax.broadcasted_iota(jnp.int32, (_L, _L), 1)
    LT = (ii < jj).astype(jnp.float32)
    ones_col = jnp.ones((_L, 1), jnp.float32)

    xw_flat = xw_ref[0].reshape(_WS, _WPB, _WS, _DIM).transpose(
        1, 0, 2, 3).reshape(FL, _DIM)
    rawT = jax.lax.dot_general(
        rW_ref[...], xw_flat, (((0,), (1,)), ((), ())),
        preferred_element_type=jnp.float32)
    logitsT = jax.nn.sigmoid(rawT)
    lT = logitsT.reshape(NR, _L)
    qkvT = jax.lax.dot_general(
        wqkv_ref[...], xw_flat, (((0,), (1,)), ((), ())),
        preferred_element_type=jnp.float32)

    v = jnp.where(jlane == 0, -1.0, lT)
    size = 2
    while size <= _L:
        desc = (jlane & size) == 0
        stride = size // 2
        while stride >= 1:
            is_low = (jlane & stride) == 0
            pv = jnp.where(is_low, pltpu.roll(v, _L - stride, 1),
                           pltpu.roll(v, stride, 1))
            keep_max = desc == is_low
            v = jnp.where(keep_max, jnp.maximum(v, pv), jnp.minimum(v, pv))
            stride //= 2
        size *= 2
    tau = v[:, _K1 - 1:_K1]

    tail = jlane >= 1
    gt = (lT > tau) & tail
    eq = (lT == tau) & tail
    n_gt = jnp.dot(gt.astype(jnp.float32), ones_col,
                   preferred_element_type=jnp.float32)
    prefix = jnp.dot(eq.astype(jnp.float32), LT,
                     preferred_element_type=jnp.float32)
    mask = (jlane == 0) | gt | (eq & (prefix < (_K1 - n_gt)))
    maskf = mask.astype(jnp.float32)
    gatef = jnp.where(mask, lT, 0.0)

    pairs = [(w, h) for w in range(_WPB) for h in range(_HEADS)]
    qkvb = qkvT.astype(jnp.bfloat16)
    maskb = maskf.astype(jnp.bfloat16)
    ps, vms = [], []
    for w, h in pairs:
        base = w * _L
        qT = qkvb[h * 48:h * 48 + 16, base:base + _L]
        kT = qkvb[h * 48 + 16:h * 48 + 32, base:base + _L]
        vT = qkvb[h * 48 + 32:h * 48 + 48, base:base + _L]
        r = h * _WPB + w
        mrow = maskb[r:r + 1, :]
        scores = jax.lax.dot_general(
            kT, qT, (((0,), (0,)), ((), ())),
            preferred_element_type=jnp.float32)
        ps.append(jnp.exp(scores).astype(jnp.bfloat16))
        vms.append(jnp.concatenate([vT * mrow, mrow], axis=0))
    oTs = {}
    for i, (w, h) in enumerate(pairs):
        r = h * _WPB + w
        os_ = jax.lax.dot_general(
            vms[i], ps[i], (((1,), (0,)), ((), ())),
            preferred_element_type=jnp.float32)
        oTs[(w, h)] = os_[:_HD] * (gatef[r:r + 1, :] / os_[_HD:_HD + 1])

    ocatT = jnp.concatenate(
        [jnp.concatenate([oTs[(w, h)] for h in range(_HEADS)], axis=0)
         for w in range(_WPB)], axis=1)
    presT = jnp.dot(woT_ref[...], ocatT,
                    preferred_element_type=jnp.float32)
    y = jax.lax.dot_general(
        presT, pW_ref[...], (((0,), (0,)), ((), ())),
        preferred_element_type=jnp.float32) + pb_ref[...]
    out_ref[0] = y.reshape(_WPB, _WS, _WS, _DIM).transpose(
        1, 0, 2, 3).reshape(_WS, _WPB * _WS, _DIM)


def kernel(x, r_W, W_qkv, W_o, proj_W, proj_b):
    B, H, W, C = x.shape
    nh = H // _WS

    qscale = jnp.concatenate([jnp.full((_HD,), _SCALE, jnp.float32),
                              jnp.ones((2 * _HD,), jnp.float32)])
    wqkv = jnp.transpose(W_qkv * qscale, (1, 0, 2)).reshape(C, _HEADS * 3 * _HD)
    woT = W_o.reshape(_HEADS * _HD, C).T
    pb = proj_b.reshape(1, C)

    return pl.pallas_call(
        _wmosa_body,
        grid=(B * nh,),
        in_specs=[
            pl.BlockSpec((1, _WS, W, C), lambda i: (i // nh, i % nh, 0, 0)),
            pl.BlockSpec((C, _HEADS), lambda i: (0, 0)),
            pl.BlockSpec((C, _HEADS * 3 * _HD), lambda i: (0, 0)),
            pl.BlockSpec((C, _HEADS * _HD), lambda i: (0, 0)),
            pl.BlockSpec((C, C), lambda i: (0, 0)),
            pl.BlockSpec((1, C), lambda i: (0, 0)),
        ],
        out_specs=pl.BlockSpec((1, _WS, W, C), lambda i: (i // nh, i % nh, 0, 0)),
        out_shape=jax.ShapeDtypeStruct((B, H, W, C), x.dtype),
    )(x, r_W, wqkv, woT, proj_W, pb)
